# baseline (speedup 1.0000x reference)
import jax
import jax.numpy as jnp
from jax import lax
from jax.experimental import pallas as pl
from jax.experimental.pallas import tpu as pltpu
from jax.experimental.pallas import tpu_sc as plsc

_B = 16384
_C = 1000
_K = 5
_BLK = 512

_NC, _NS, _L = 2, 16, 16
_NW = _NC * _NS
_RW = _B // _NW
_NIDX = _K * _RW


def _dense_kernel(x_ref, o_ref):
    i = pl.program_id(0)
    x = x_ref[...]
    s = jnp.sum(jnp.maximum(x, 0.0) + jnp.log1p(jnp.exp(-jnp.abs(x))))

    @pl.when(i == 0)
    def _init():
        o_ref[...] = jnp.zeros((1, 1), jnp.float32)

    o_ref[...] += s.reshape(1, 1)


def _dense_sum(x):
    return pl.pallas_call(
        _dense_kernel,
        grid=(_B // _BLK,),
        in_specs=[pl.BlockSpec((_BLK, _C), lambda i: (i, 0))],
        out_specs=pl.BlockSpec((1, 1), lambda i: (0, 0)),
        out_shape=jax.ShapeDtypeStruct((1, 1), jnp.float32),
    )(x)[0, 0]


def _sc_gather_kernel(xf_hbm, yt_hbm, out_hbm, y0_v, y1_v, y2_v, y3_v,
                      y4_v, idx_v, wgt_v, vals_v, acc_v):
    ycols = (y0_v, y1_v, y2_v, y3_v, y4_v)
    wid = lax.axis_index("s") * _NC + lax.axis_index("c")
    base = wid * _RW

    for k in range(_K):
        pltpu.sync_copy(yt_hbm.at[pl.ds(k * _B + base, _RW)], ycols[k])

    lanes = lax.iota(jnp.int32, _L)
    for i in range(_RW // _L):
        rows = (base + i * _L) + lanes
        ys = []
        for k in range(_K):
            yk = ycols[k][pl.ds(i * _L, _L)]
            valid = yk < _C
            for d in range(k):
                valid = valid & (yk != ys[d])
            ys.append(yk)
            pos = k * _RW + i * _L
            idx_v[pl.ds(pos, _L)] = rows * _C + jnp.minimum(yk, _C - 1)
            wgt_v[pl.ds(pos, _L)] = jnp.where(valid, 1.0, 0.0)

    pltpu.sync_copy(xf_hbm.at[idx_v], vals_v)

    acc = jnp.zeros((_L,), jnp.float32)
    for j in range(_NIDX // _L):
        pos = j * _L
        acc = acc + vals_v[pl.ds(pos, _L)] * wgt_v[pl.ds(pos, _L)]
    acc_v[...] = acc
    pltpu.sync_copy(acc_v, out_hbm.at[wid])


def _sc_gather_sum(x_flat, y_t):
    mesh = plsc.VectorSubcoreMesh(core_axis_name="c", subcore_axis_name="s")
    call = pl.kernel(
        _sc_gather_kernel,
        out_type=jax.ShapeDtypeStruct((_NW, _L), jnp.float32),
        mesh=mesh,
        scratch_types=[
            pltpu.VMEM((_RW,), jnp.int32),
            pltpu.VMEM((_RW,), jnp.int32),
            pltpu.VMEM((_RW,), jnp.int32),
            pltpu.VMEM((_RW,), jnp.int32),
            pltpu.VMEM((_RW,), jnp.int32),
            pltpu.VMEM((_NIDX,), jnp.int32),
            pltpu.VMEM((_NIDX,), jnp.float32),
            pltpu.VMEM((_NIDX,), jnp.float32),
            pltpu.VMEM((_L,), jnp.float32),
        ],
    )
    return call(x_flat, y_t)


def kernel(out, y_inds):
    partials = _sc_gather_sum(
        out.reshape(-1),
        jnp.transpose(y_inds.astype(jnp.int32)).reshape(-1))
    dense = _dense_sum(out)
    loss = (dense - jnp.sum(partials)) / (_B * _C)
    return loss.astype(out.dtype)
